# baseline (speedup 1.0000x reference)
import functools

import jax
import jax.numpy as jnp
from jax import lax
from jax.experimental import pallas as pl
from jax.experimental.pallas import tpu as pltpu
from jax.experimental.pallas import tpu_sc as plsc

BLANK_ID = 140
NUM_CLASSES = 141
BATCH = 64
SEQ = 4096

N_SC = 16
N_TC = BATCH - N_SC

NUM_CORES = 2
NUM_SUBCORES = 16
LANES = 16
NUM_WORKERS = NUM_CORES * NUM_SUBCORES
FRAMES_PER_WORKER = N_SC * SEQ // NUM_WORKERS
CHUNK = 256
CHUNKS_PER_ROW = SEQ // CHUNK
NUM_CHUNKS = FRAMES_PER_WORKER // CHUNK
CHUNK_WORDS = CHUNK * NUM_CLASSES
GROUPS = CHUNK // LANES

assert FRAMES_PER_WORKER % CHUNK == 0 and NUM_CHUNKS % 2 == 0

_mesh = plsc.VectorSubcoreMesh(
    core_axis_name="c",
    subcore_axis_name="s",
    num_cores=NUM_CORES,
    num_subcores=NUM_SUBCORES,
)


@functools.partial(
    pl.kernel,
    out_type=(
        jax.ShapeDtypeStruct((N_SC * SEQ,), jnp.int32),
        jax.ShapeDtypeStruct((N_SC * SEQ,), jnp.int32),
    ),
    mesh=_mesh,
    compiler_params=pltpu.CompilerParams(needs_layout_passes=False),
    scratch_types=[
        pltpu.VMEM((CHUNK_WORDS,), jnp.float32),
        pltpu.VMEM((CHUNK_WORDS,), jnp.float32),
        pltpu.VMEM((LANES + CHUNK,), jnp.int32),
        pltpu.VMEM((CHUNK,), jnp.int32),
        pltpu.SemaphoreType.DMA,
        pltpu.SemaphoreType.DMA,
    ],
)
def _ctc_sc(probs_hbm, idx_hbm, keep_hbm, buf0, buf1, idxbuf, keepbuf, sem0, sem1):
    wid = lax.axis_index("s") * NUM_CORES + lax.axis_index("c")
    frame_base = wid * FRAMES_PER_WORKER
    word_base = frame_base * NUM_CLASSES

    lanes_iota = lax.iota(jnp.int32, LANES)
    group_stride = lanes_iota * NUM_CLASSES
    minus_one = jnp.full((LANES,), -1, jnp.int32)
    one = jnp.full((LANES,), 1, jnp.int32)

    def compute_chunk(k, buf, carry_vec):
        row_start = ((wid * NUM_CHUNKS + k) % CHUNKS_PER_ROW) == 0
        pre = jnp.where(row_start, minus_one, carry_vec)
        idxbuf[pl.ds(0, LANES)] = pre

        def group(g, carry):
            gbase = g * (LANES * NUM_CLASSES)
            gidx0 = gbase + group_stride
            best = plsc.load_gather(buf, [gidx0])
            gbest = gidx0
            gidx = gidx0
            for c in range(1, NUM_CLASSES):
                gidx = gidx + one
                v = plsc.load_gather(buf, [gidx])
                m = v > best
                best = jnp.maximum(best, v)
                gbest = jnp.where(m, gidx, gbest)
            besti = gbest - gidx0
            idxbuf[pl.ds(LANES + g * LANES, LANES)] = besti
            prev = plsc.load_gather(idxbuf, [(LANES - 1) + g * LANES + lanes_iota])
            keep = ((besti != prev) & (besti != BLANK_ID)).astype(jnp.int32)
            keepbuf[pl.ds(g * LANES, LANES)] = keep
            return besti

        last = lax.fori_loop(0, GROUPS, group, pre)

        out_base = frame_base + k * CHUNK
        pltpu.sync_copy(idxbuf.at[pl.ds(LANES, CHUNK)], idx_hbm.at[pl.ds(out_base, CHUNK)])
        pltpu.sync_copy(keepbuf, keep_hbm.at[pl.ds(out_base, CHUNK)])
        return last

    def in_slice(k):
        return probs_hbm.at[pl.ds(word_base + k * CHUNK_WORDS, CHUNK_WORDS)]

    pltpu.async_copy(in_slice(0), buf0, sem0)

    def body(i, carry_vec):
        k0 = 2 * i
        pltpu.async_copy(in_slice(k0 + 1), buf1, sem1)
        pltpu.make_async_copy(in_slice(k0), buf0, sem0).wait()
        carry_vec = compute_chunk(k0, buf0, carry_vec)

        @pl.when(k0 + 2 < NUM_CHUNKS)
        def _():
            pltpu.async_copy(in_slice(k0 + 2), buf0, sem0)

        pltpu.make_async_copy(in_slice(k0 + 1), buf1, sem1).wait()
        carry_vec = compute_chunk(k0 + 1, buf1, carry_vec)
        return carry_vec

    lax.fori_loop(0, NUM_CHUNKS // 2, body, minus_one)


PAD = 128


def _tc_body(probs_hbm, idx_ref, keep_ref, buf, shift_buf, sem):
    i = pl.program_id(0)
    slot = lax.rem(i, 2)
    nslot = lax.rem(i + 1, 2)

    def start(row, s):
        pltpu.make_async_copy(probs_hbm.at[N_SC + row], buf.at[s], sem.at[s]).start()

    @pl.when(i == 0)
    def _():
        shift_buf[pl.ds(0, PAD)] = jnp.full((PAD,), -1, jnp.int32)
        start(0, 0)

    @pl.when(i + 1 < N_TC)
    def _():
        start(i + 1, nslot)

    pltpu.make_async_copy(probs_hbm.at[N_SC + i], buf.at[slot], sem.at[slot]).wait()

    xt = buf[slot].T
    m = jnp.max(xt, axis=0)
    idx = jnp.argmax(xt, axis=0)

    shift_buf[pl.ds(PAD, SEQ)] = idx
    prev = shift_buf[pl.ds(PAD - 1, SEQ)]
    keep = ((idx != prev) & (idx != BLANK_ID)).astype(jnp.int32)

    r = lax.rem(i, 8)
    idx_ref[pl.ds(r, 1), :] = idx.reshape(1, SEQ)
    keep_ref[pl.ds(r, 1), :] = keep.reshape(1, SEQ)


def _tc_call(probs):
    return pl.pallas_call(
        _tc_body,
        grid=(N_TC,),
        in_specs=[pl.BlockSpec(memory_space=pl.ANY)],
        out_specs=[
            pl.BlockSpec((8, SEQ), lambda i: (i // 8, 0)),
            pl.BlockSpec((8, SEQ), lambda i: (i // 8, 0)),
        ],
        out_shape=[
            jax.ShapeDtypeStruct((N_TC, SEQ), jnp.int32),
            jax.ShapeDtypeStruct((N_TC, SEQ), jnp.int32),
        ],
        scratch_shapes=[
            pltpu.VMEM((2, SEQ, NUM_CLASSES), jnp.float32),
            pltpu.VMEM((PAD + SEQ,), jnp.int32),
            pltpu.SemaphoreType.DMA((2,)),
        ],
    )(probs)


def kernel(probs):
    sc_flat = probs[:N_SC].reshape(-1)
    tc_idx, tc_keep = _tc_call(probs)
    sc_idx, sc_keep = _ctc_sc(sc_flat)
    idx = jnp.concatenate([sc_idx.reshape(N_SC, SEQ), tc_idx], axis=0)
    keep = jnp.concatenate([sc_keep.reshape(N_SC, SEQ), tc_keep], axis=0)
    return idx, keep.astype(bool)
